# initial kernel scaffold (unmeasured)
import jax
import jax.numpy as jnp
from jax import lax
from jax.experimental import pallas as pl
from jax.experimental.pallas import tpu as pltpu


def kernel(x, dy):
    bl, d = x.shape
    _, f = dy.shape
    blk = d // 4

    xi = lax.axis_index("x")
    yi = lax.axis_index("y")
    zi = lax.axis_index("z")
    r = 2 * yi + zi

    x_slice = lax.dynamic_slice(x, (0, r * blk), (bl, blk))
    p = lax.dot_general(
        x_slice, dy, (((0,), (0,)), ((), ())),
        preferred_element_type=jnp.float32,
    )

    def body(p_ref, out_ref, xrecv_ref,
             x_send_sem, x_recv_sem, y_send_sem, y_recv_sem,
             z_send_sem, z_recv_sem, copy_sem):
        xi = lax.axis_index("x")
        yi = lax.axis_index("y")
        zi = lax.axis_index("z")
        is_builder = yi == xi

        barrier = pltpu.get_barrier_semaphore()
        for nbr in ((1 - xi, yi, zi), (xi, 1 - yi, zi), (xi, yi, 1 - zi)):
            pl.semaphore_signal(
                barrier, inc=1, device_id=nbr,
                device_id_type=pl.DeviceIdType.MESH,
            )
        pl.semaphore_wait(barrier, 3)

        x_rdma = pltpu.make_async_remote_copy(
            src_ref=p_ref, dst_ref=xrecv_ref,
            send_sem=x_send_sem, recv_sem=x_recv_sem,
            device_id=(1 - xi, yi, zi),
            device_id_type=pl.DeviceIdType.MESH,
        )

        @pl.when(jnp.logical_not(is_builder))
        def _():
            x_rdma.start()
            x_rdma.wait_send()

        @pl.when(is_builder)
        def _():
            x_rdma.wait_recv()
            xrecv_ref[...] = p_ref[...] + xrecv_ref[...]
            cp = pltpu.make_async_copy(
                xrecv_ref, out_ref.at[pl.ds(zi * blk, blk)], copy_sem)
            cp.start()
            cp.wait()

        y_rdma = pltpu.make_async_remote_copy(
            src_ref=xrecv_ref,
            dst_ref=out_ref.at[pl.ds(zi * blk, blk)],
            send_sem=y_send_sem, recv_sem=y_recv_sem,
            device_id=(xi, 1 - yi, zi),
            device_id_type=pl.DeviceIdType.MESH,
        )

        @pl.when(is_builder)
        def _():
            y_rdma.start()
            y_rdma.wait_send()

        @pl.when(jnp.logical_not(is_builder))
        def _():
            y_rdma.wait_recv()

        z_rdma = pltpu.make_async_remote_copy(
            src_ref=out_ref.at[pl.ds(zi * blk, blk)],
            dst_ref=out_ref.at[pl.ds(zi * blk, blk)],
            send_sem=z_send_sem, recv_sem=z_recv_sem,
            device_id=(xi, yi, 1 - zi),
            device_id_type=pl.DeviceIdType.MESH,
        )
        z_rdma.start()
        z_rdma.wait()

    return pl.pallas_call(
        body,
        out_shape=jax.ShapeDtypeStruct((d // 2, f), jnp.float32),
        in_specs=[pl.BlockSpec(memory_space=pltpu.VMEM)],
        out_specs=pl.BlockSpec(memory_space=pltpu.ANY),
        scratch_shapes=[
            pltpu.VMEM((blk, f), jnp.float32),
            pltpu.SemaphoreType.DMA,
            pltpu.SemaphoreType.DMA,
            pltpu.SemaphoreType.DMA,
            pltpu.SemaphoreType.DMA,
            pltpu.SemaphoreType.DMA,
            pltpu.SemaphoreType.DMA,
            pltpu.SemaphoreType.DMA,
        ],
        compiler_params=pltpu.CompilerParams(
            collective_id=0, has_side_effects=True,
        ),
    )(p)


# baseline (device time: 1283128 ns/iter reference)
import jax
import jax.numpy as jnp
from jax import lax
from jax.experimental import pallas as pl
from jax.experimental.pallas import tpu as pltpu


def kernel(x, dy):
    bl, d = x.shape
    _, f = dy.shape
    blk = d // 4

    xi = lax.axis_index("x")
    yi = lax.axis_index("y")
    zi = lax.axis_index("z")
    r = 2 * yi + zi

    x_slice = lax.dynamic_slice(x, (0, r * blk), (bl, blk))
    p = lax.dot_general(
        x_slice, dy, (((0,), (0,)), ((), ())),
        preferred_element_type=jnp.float32,
    )

    cw = f // 4

    def body(p_ref, out_ref, va, vb,
             x_send_sem, x_recv_sem, y_send_sem, y_recv_sem,
             z_send_sem, z_recv_sem, z_credit, copy_sem_a, copy_sem_b):
        xi = lax.axis_index("x")
        yi = lax.axis_index("y")
        zi = lax.axis_index("z")
        is_builder = yi == xi

        barrier = pltpu.get_barrier_semaphore()
        for nbr in ((1 - xi, yi, zi), (xi, 1 - yi, zi), (xi, yi, 1 - zi)):
            pl.semaphore_signal(
                barrier, inc=1, device_id=nbr,
                device_id_type=pl.DeviceIdType.MESH,
            )
        pl.semaphore_wait(barrier, 3)

        stage = out_ref.at[pl.ds((1 - zi) * blk, blk)]
        x_rdma = pltpu.make_async_remote_copy(
            src_ref=p_ref, dst_ref=stage,
            send_sem=x_send_sem, recv_sem=x_recv_sem,
            device_id=(1 - xi, yi, zi),
            device_id_type=pl.DeviceIdType.MESH,
        )

        @pl.when(jnp.logical_not(is_builder))
        def _():
            x_rdma.start()
            x_rdma.wait_send()
            pl.semaphore_signal(
                z_credit, inc=1, device_id=(xi, yi, 1 - zi),
                device_id_type=pl.DeviceIdType.MESH,
            )

        @pl.when(is_builder)
        def _():
            x_rdma.wait_recv()
            for j in range(f // cw):
                cols = pl.ds(j * cw, cw)
                ca = pltpu.make_async_copy(
                    p_ref.at[:, cols], va, copy_sem_a)
                cb = pltpu.make_async_copy(
                    stage.at[:, cols], vb, copy_sem_b)
                ca.start()
                cb.start()
                ca.wait()
                cb.wait()
                va[...] = va[...] + vb[...]
                co = pltpu.make_async_copy(
                    va, out_ref.at[pl.ds(zi * blk, blk), cols], copy_sem_a)
                co.start()
                co.wait()
            pl.semaphore_signal(
                z_credit, inc=1, device_id=(xi, yi, 1 - zi),
                device_id_type=pl.DeviceIdType.MESH,
            )

        y_rdma = pltpu.make_async_remote_copy(
            src_ref=out_ref.at[pl.ds(zi * blk, blk)],
            dst_ref=out_ref.at[pl.ds(zi * blk, blk)],
            send_sem=y_send_sem, recv_sem=y_recv_sem,
            device_id=(xi, 1 - yi, zi),
            device_id_type=pl.DeviceIdType.MESH,
        )

        @pl.when(is_builder)
        def _():
            y_rdma.start()
            y_rdma.wait_send()

        @pl.when(jnp.logical_not(is_builder))
        def _():
            y_rdma.wait_recv()

        z_rdma = pltpu.make_async_remote_copy(
            src_ref=out_ref.at[pl.ds(zi * blk, blk)],
            dst_ref=out_ref.at[pl.ds(zi * blk, blk)],
            send_sem=z_send_sem, recv_sem=z_recv_sem,
            device_id=(xi, yi, 1 - zi),
            device_id_type=pl.DeviceIdType.MESH,
        )
        pl.semaphore_wait(z_credit, 1)
        z_rdma.start()
        z_rdma.wait()

    return pl.pallas_call(
        body,
        out_shape=jax.ShapeDtypeStruct((d // 2, f), jnp.float32),
        in_specs=[pl.BlockSpec(memory_space=pl.ANY)],
        out_specs=pl.BlockSpec(memory_space=pl.ANY),
        scratch_shapes=[
            pltpu.VMEM((blk, cw), jnp.float32),
            pltpu.VMEM((blk, cw), jnp.float32),
            pltpu.SemaphoreType.DMA,
            pltpu.SemaphoreType.DMA,
            pltpu.SemaphoreType.DMA,
            pltpu.SemaphoreType.DMA,
            pltpu.SemaphoreType.DMA,
            pltpu.SemaphoreType.DMA,
            pltpu.SemaphoreType.REGULAR,
            pltpu.SemaphoreType.DMA,
            pltpu.SemaphoreType.DMA,
        ],
        compiler_params=pltpu.CompilerParams(
            collective_id=0, has_side_effects=True,
        ),
    )(p)


# device time: 620688 ns/iter; 2.0673x vs baseline; 2.0673x over previous
import jax
import jax.numpy as jnp
from jax import lax
from jax.experimental import pallas as pl
from jax.experimental.pallas import tpu as pltpu

C = 8


def kernel(x, dy):
    bl, d = x.shape
    _, f = dy.shape
    blk = d // 4
    cw = f // C

    xi = lax.axis_index("x")
    yi = lax.axis_index("y")
    zi = lax.axis_index("z")
    r = 2 * yi + zi

    x_slice = lax.dynamic_slice(x, (0, r * blk), (bl, blk))
    p = lax.dot_general(
        x_slice, dy, (((0,), (0,)), ((), ())),
        preferred_element_type=jnp.float32,
    )

    def body(p_ref, out_ref, va, vb,
             x_send, x_recv, y_send, y_recv, z_send, z_recv,
             z_credit, copy_a, copy_b):
        xi = lax.axis_index("x")
        yi = lax.axis_index("y")
        zi = lax.axis_index("z")
        is_builder = yi == xi

        barrier = pltpu.get_barrier_semaphore()
        for nbr in ((1 - xi, yi, zi), (xi, 1 - yi, zi), (xi, yi, 1 - zi)):
            pl.semaphore_signal(
                barrier, inc=1, device_id=nbr,
                device_id_type=pl.DeviceIdType.MESH,
            )
        pl.semaphore_wait(barrier, 3)

        mine = out_ref.at[pl.ds(zi * blk, blk)]
        stage = out_ref.at[pl.ds((1 - zi) * blk, blk)]

        def cols(c):
            return pl.ds(c * cw, cw)

        def x_rdma(c):
            return pltpu.make_async_remote_copy(
                src_ref=p_ref.at[:, cols(c)], dst_ref=stage.at[:, cols(c)],
                send_sem=x_send.at[c], recv_sem=x_recv.at[c],
                device_id=(1 - xi, yi, zi),
                device_id_type=pl.DeviceIdType.MESH,
            )

        def y_rdma(c):
            return pltpu.make_async_remote_copy(
                src_ref=mine.at[:, cols(c)], dst_ref=mine.at[:, cols(c)],
                send_sem=y_send.at[c], recv_sem=y_recv.at[c],
                device_id=(xi, 1 - yi, zi),
                device_id_type=pl.DeviceIdType.MESH,
            )

        def z_rdma(c):
            return pltpu.make_async_remote_copy(
                src_ref=mine.at[:, cols(c)], dst_ref=mine.at[:, cols(c)],
                send_sem=z_send.at[c], recv_sem=z_recv.at[c],
                device_id=(xi, yi, 1 - zi),
                device_id_type=pl.DeviceIdType.MESH,
            )

        @pl.when(jnp.logical_not(is_builder))
        def _():
            for c in range(C):
                x_rdma(c).start()
            pl.semaphore_signal(
                z_credit, inc=C, device_id=(xi, yi, 1 - zi),
                device_id_type=pl.DeviceIdType.MESH,
            )
            for c in range(C):
                y_rdma(c).wait_recv()
                pl.semaphore_wait(z_credit, 1)
                z_rdma(c).start()
            for c in range(C):
                x_rdma(c).wait_send()

        @pl.when(is_builder)
        def _():
            for c in range(C):
                x_rdma(c).wait_recv()
                ca = pltpu.make_async_copy(p_ref.at[:, cols(c)], va, copy_a)
                cb = pltpu.make_async_copy(stage.at[:, cols(c)], vb, copy_b)
                ca.start()
                cb.start()
                ca.wait()
                cb.wait()
                va[...] = va[...] + vb[...]
                co = pltpu.make_async_copy(va, mine.at[:, cols(c)], copy_a)
                co.start()
                co.wait()
                pl.semaphore_signal(
                    z_credit, inc=1, device_id=(xi, yi, 1 - zi),
                    device_id_type=pl.DeviceIdType.MESH,
                )
                y_rdma(c).start()
                pl.semaphore_wait(z_credit, 1)
                z_rdma(c).start()
            for c in range(C):
                y_rdma(c).wait_send()

        for c in range(C):
            z_rdma(c).wait_send()
            z_rdma(c).wait_recv()

    return pl.pallas_call(
        body,
        out_shape=jax.ShapeDtypeStruct((d // 2, f), jnp.float32),
        in_specs=[pl.BlockSpec(memory_space=pl.ANY)],
        out_specs=pl.BlockSpec(memory_space=pl.ANY),
        scratch_shapes=[
            pltpu.VMEM((blk, cw), jnp.float32),
            pltpu.VMEM((blk, cw), jnp.float32),
            pltpu.SemaphoreType.DMA((C,)),
            pltpu.SemaphoreType.DMA((C,)),
            pltpu.SemaphoreType.DMA((C,)),
            pltpu.SemaphoreType.DMA((C,)),
            pltpu.SemaphoreType.DMA((C,)),
            pltpu.SemaphoreType.DMA((C,)),
            pltpu.SemaphoreType.REGULAR,
            pltpu.SemaphoreType.DMA,
            pltpu.SemaphoreType.DMA,
        ],
        compiler_params=pltpu.CompilerParams(
            collective_id=0, has_side_effects=True,
        ),
    )(p)
